# baseline (speedup 1.0000x reference)
import functools

import jax
import jax.numpy as jnp
from jax.experimental import pallas as pl
from jax.experimental.pallas import tpu as pltpu

N_TOK = 36864
N_CODES = 1024
DIM = 64
BLK = 512


def _split2(a):
    hi = a.astype(jnp.bfloat16)
    lo = (a - hi.astype(jnp.float32)).astype(jnp.bfloat16)
    return hi, lo


def _split3(a):
    hi = a.astype(jnp.bfloat16)
    r = a - hi.astype(jnp.float32)
    mid = r.astype(jnp.bfloat16)
    lo = (r - mid.astype(jnp.float32)).astype(jnp.bfloat16)
    return hi, mid, lo


def _mm(a, b):
    return jax.lax.dot_general(
        a, b, (((1,), (0,)), ((), ())),
        preferred_element_type=jnp.float32)


def _ref_order_sum(sq):
    t = sq[:, 0:8]
    for i in range(1, 8):
        t = t + sq[:, 8 * i:8 * i + 8]
    u = t[:, 0:4] + t[:, 4:8]
    w = u[:, 0:2] + u[:, 2:4]
    return w[:, 0:1] + w[:, 1:2]


def _prep_kernel(cbt2_ref, cb_ref, ch_ref, cl_ref,
                 cbh_ref, cbm_ref, cbl_ref, c2_ref):
    ch, cl = _split2(cbt2_ref[...])
    ch_ref[...] = ch
    cl_ref[...] = cl
    cbh, cbm, cbl = _split3(cb_ref[...])
    cbh_ref[...] = cbh
    cbm_ref[...] = cbm
    cbl_ref[...] = cbl
    cbt = cbt2_ref[...] * -0.5
    c2_ref[...] = jnp.sum(cbt * cbt, axis=0, keepdims=True)


def _vq_kernel(x_ref, ch_ref, cl_ref, cbh_ref, cbm_ref, cbl_ref, c2_ref,
               l2_ref, codes_ref, vec_ref):
    x = x_ref[...]
    xh, xl = _split2(x)
    ch = ch_ref[...]
    cross = _mm(xh, ch) + _mm(xh, cl_ref[...]) + _mm(xl, ch)
    e = c2_ref[...] + cross
    x2 = jnp.sum(x * x, axis=1, keepdims=True)
    l2_ref[...] = x2 + e

    iota = jax.lax.broadcasted_iota(jnp.int32, (1, N_CODES), 1)
    am1 = jnp.argmin(e, axis=1).astype(jnp.int32)
    oh1 = am1[:, None] == iota
    e2 = jnp.where(oh1, jnp.float32(jnp.inf), e)

    def gather_exact(oh):
        ohb = oh.astype(jnp.bfloat16)
        return (_mm(ohb, cbh_ref[...]) + _mm(ohb, cbm_ref[...])
                + _mm(ohb, cbl_ref[...]))

    g0 = gather_exact(oh1)

    m1v = jnp.min(e, axis=1, keepdims=True)
    m2v = jnp.min(e2, axis=1, keepdims=True)
    tight = jnp.min(m2v - m1v) <= jnp.float32(1e-4)

    @pl.when(jnp.logical_not(tight))
    def _fast():
        codes_ref[...] = am1
        vec_ref[...] = g0

    @pl.when(tight)
    def _exact_tiebreak():
        am2 = jnp.argmin(e2, axis=1).astype(jnp.int32)
        oh2 = am2[:, None] == iota
        g1 = gather_exact(oh2)
        d0 = _ref_order_sum((g0 - x) * (g0 - x))
        d1 = _ref_order_sum((g1 - x) * (g1 - x))
        a1 = am1[:, None]
        a2 = am2[:, None]
        take2 = (d1 < d0) | ((d1 == d0) & (a2 < a1))
        codes_ref[...] = jnp.where(take2, a2, a1)[:, 0]
        vec_ref[...] = jnp.where(take2, g1, g0)


@functools.partial(jax.jit, static_argnames=())
def kernel(x, codebook):
    bf16 = jnp.bfloat16
    f32 = jnp.float32
    cbt2 = -2.0 * codebook.T
    ch, cl, cbh, cbm, cbl, c2 = pl.pallas_call(
        _prep_kernel,
        out_shape=[
            jax.ShapeDtypeStruct((DIM, N_CODES), bf16),
            jax.ShapeDtypeStruct((DIM, N_CODES), bf16),
            jax.ShapeDtypeStruct((N_CODES, DIM), bf16),
            jax.ShapeDtypeStruct((N_CODES, DIM), bf16),
            jax.ShapeDtypeStruct((N_CODES, DIM), bf16),
            jax.ShapeDtypeStruct((1, N_CODES), f32),
        ],
    )(cbt2, codebook)

    grid = (N_TOK // BLK,)
    l2, codes, vec = pl.pallas_call(
        _vq_kernel,
        grid=grid,
        in_specs=[
            pl.BlockSpec((BLK, DIM), lambda i: (i, 0)),
            pl.BlockSpec((DIM, N_CODES), lambda i: (0, 0)),
            pl.BlockSpec((DIM, N_CODES), lambda i: (0, 0)),
            pl.BlockSpec((N_CODES, DIM), lambda i: (0, 0)),
            pl.BlockSpec((N_CODES, DIM), lambda i: (0, 0)),
            pl.BlockSpec((N_CODES, DIM), lambda i: (0, 0)),
            pl.BlockSpec((1, N_CODES), lambda i: (0, 0)),
        ],
        out_specs=[
            pl.BlockSpec((BLK, N_CODES), lambda i: (i, 0)),
            pl.BlockSpec((BLK,), lambda i: (i,)),
            pl.BlockSpec((BLK, DIM), lambda i: (i, 0)),
        ],
        out_shape=[
            jax.ShapeDtypeStruct((N_TOK, N_CODES), f32),
            jax.ShapeDtypeStruct((N_TOK,), jnp.int32),
            jax.ShapeDtypeStruct((N_TOK, DIM), f32),
        ],
        compiler_params=pltpu.CompilerParams(
            dimension_semantics=("parallel",)),
    )(x, ch, cl, cbh, cbm, cbl, c2)
    return (vec, codes, l2)


# speedup vs baseline: 1.3125x; 1.3125x over previous
import functools

import jax
import jax.numpy as jnp
from jax.experimental import pallas as pl
from jax.experimental.pallas import tpu as pltpu

N_TOK = 36864
N_CODES = 1024
DIM = 64
BLK = 512


def _split2(a):
    hi = a.astype(jnp.bfloat16)
    lo = (a - hi.astype(jnp.float32)).astype(jnp.bfloat16)
    return hi, lo


def _split3(a):
    hi = a.astype(jnp.bfloat16)
    r = a - hi.astype(jnp.float32)
    mid = r.astype(jnp.bfloat16)
    lo = (r - mid.astype(jnp.float32)).astype(jnp.bfloat16)
    return hi, mid, lo


def _mm(a, b):
    return jax.lax.dot_general(
        a, b, (((1,), (0,)), ((), ())),
        preferred_element_type=jnp.float32)


def _ref_order_sum_t(sq):
    t = sq[0:8, :]
    for i in range(1, 8):
        t = t + sq[8 * i:8 * i + 8, :]
    u = t[0:4, :] + t[4:8, :]
    w = u[0:2, :] + u[2:4, :]
    return w[0:1, :] + w[1:2, :]


def _prep_kernel(cbt2_ref, cb_ref, ch_ref, clh_ref,
                 cbh_ref, cbm_ref, cbl_ref, c2_ref):
    ch, cl = _split2(cbt2_ref[...])
    ch_ref[...] = ch
    clh_ref[0:DIM, :] = cl
    clh_ref[DIM:2 * DIM, :] = ch
    cbt = cbt2_ref[...] * -0.5
    cbh, cbm, cbl = _split3(cbt)
    cbh_ref[...] = cbh
    cbm_ref[...] = cbm
    cbl_ref[...] = cbl
    c2_ref[...] = jnp.sum(cbt * cbt, axis=0, keepdims=True)


def _mm_t(a, b):
    return jax.lax.dot_general(
        a, b, (((1,), (1,)), ((), ())),
        preferred_element_type=jnp.float32)


def _vq_kernel(x_ref, xt_ref, ch_ref, clh_ref, cbh_ref, cbm_ref, cbl_ref,
               c2_ref, l2_ref, codes_ref, vec_ref):
    x = x_ref[...]
    xh, xl = _split2(x)
    xhl = jnp.concatenate([xh, xl], axis=1)
    cross = _mm(xh, ch_ref[...]) + _mm(xhl, clh_ref[...])
    e = c2_ref[...] + cross
    x2 = jnp.sum(x * x, axis=1, keepdims=True)
    l2_ref[...] = x2 + e

    iota = jax.lax.broadcasted_iota(jnp.int32, (1, N_CODES), 1)
    am1 = jnp.argmin(e, axis=1).astype(jnp.int32)
    oh1 = am1[:, None] == iota
    e2 = jnp.where(oh1, jnp.float32(jnp.inf), e)
    am2 = jnp.argmin(e2, axis=1).astype(jnp.int32)
    oh2 = am2[:, None] == iota

    def gather_exact_t(oh):
        ohb = oh.astype(jnp.bfloat16)
        return (_mm_t(cbh_ref[...], ohb) + _mm_t(cbm_ref[...], ohb)
                + _mm_t(cbl_ref[...], ohb))

    xt = xt_ref[...]
    g0 = gather_exact_t(oh1)
    g1 = gather_exact_t(oh2)
    d0 = _ref_order_sum_t((g0 - xt) * (g0 - xt))
    d1 = _ref_order_sum_t((g1 - xt) * (g1 - xt))
    a1 = am1[None, :]
    a2 = am2[None, :]
    take2 = (d1 < d0) | ((d1 == d0) & (a2 < a1))
    codes_ref[...] = jnp.where(take2, a2, a1)[0, :]
    vec_ref[...] = jnp.where(take2, g1, g0).T


@functools.partial(jax.jit, static_argnames=())
def kernel(x, codebook):
    bf16 = jnp.bfloat16
    f32 = jnp.float32
    cbt2 = -2.0 * codebook.T
    ch, cl, cbh, cbm, cbl, c2 = pl.pallas_call(
        _prep_kernel,
        out_shape=[
            jax.ShapeDtypeStruct((DIM, N_CODES), bf16),
            jax.ShapeDtypeStruct((2 * DIM, N_CODES), bf16),
            jax.ShapeDtypeStruct((DIM, N_CODES), bf16),
            jax.ShapeDtypeStruct((DIM, N_CODES), bf16),
            jax.ShapeDtypeStruct((DIM, N_CODES), bf16),
            jax.ShapeDtypeStruct((1, N_CODES), f32),
        ],
    )(cbt2, codebook)
    xt = x.T

    grid = (N_TOK // BLK,)
    l2, codes, vec = pl.pallas_call(
        _vq_kernel,
        grid=grid,
        in_specs=[
            pl.BlockSpec((BLK, DIM), lambda i: (i, 0)),
            pl.BlockSpec((DIM, BLK), lambda i: (0, i)),
            pl.BlockSpec((DIM, N_CODES), lambda i: (0, 0)),
            pl.BlockSpec((2 * DIM, N_CODES), lambda i: (0, 0)),
            pl.BlockSpec((DIM, N_CODES), lambda i: (0, 0)),
            pl.BlockSpec((DIM, N_CODES), lambda i: (0, 0)),
            pl.BlockSpec((DIM, N_CODES), lambda i: (0, 0)),
            pl.BlockSpec((1, N_CODES), lambda i: (0, 0)),
        ],
        out_specs=[
            pl.BlockSpec((BLK, N_CODES), lambda i: (i, 0)),
            pl.BlockSpec((BLK,), lambda i: (i,)),
            pl.BlockSpec((BLK, DIM), lambda i: (i, 0)),
        ],
        out_shape=[
            jax.ShapeDtypeStruct((N_TOK, N_CODES), f32),
            jax.ShapeDtypeStruct((N_TOK,), jnp.int32),
            jax.ShapeDtypeStruct((N_TOK, DIM), f32),
        ],
        compiler_params=pltpu.CompilerParams(
            dimension_semantics=("parallel",)),
    )(x, xt, ch, cl, cbh, cbm, cbl, c2)
    return (vec, codes, l2)


# speedup vs baseline: 1.4354x; 1.0937x over previous
import functools

import jax
import jax.numpy as jnp
from jax.experimental import pallas as pl
from jax.experimental.pallas import tpu as pltpu

N_TOK = 36864
N_CODES = 1024
DIM = 64
BLK = 1024


def _split2(a):
    hi = a.astype(jnp.bfloat16)
    lo = (a - hi.astype(jnp.float32)).astype(jnp.bfloat16)
    return hi, lo


def _split3(a):
    hi = a.astype(jnp.bfloat16)
    r = a - hi.astype(jnp.float32)
    mid = r.astype(jnp.bfloat16)
    lo = (r - mid.astype(jnp.float32)).astype(jnp.bfloat16)
    return hi, mid, lo


def _mm(a, b):
    return jax.lax.dot_general(
        a, b, (((1,), (0,)), ((), ())),
        preferred_element_type=jnp.float32)


def _ref_order_sum_t(sq):
    t = sq[0:8, :]
    for i in range(1, 8):
        t = t + sq[8 * i:8 * i + 8, :]
    u = t[0:4, :] + t[4:8, :]
    w = u[0:2, :] + u[2:4, :]
    return w[0:1, :] + w[1:2, :]


def _prep_kernel(cbt2_ref, cb_ref, ch_ref, clh_ref,
                 cbh_ref, cbm_ref, cbl_ref, c2_ref):
    ch, cl = _split2(cbt2_ref[...])
    ch_ref[...] = ch
    clh_ref[0:DIM, :] = cl
    clh_ref[DIM:2 * DIM, :] = ch
    cbt = cbt2_ref[...] * -0.5
    cbh, cbm, cbl = _split3(cbt)
    cbh_ref[...] = cbh
    cbm_ref[...] = cbm
    cbl_ref[...] = cbl
    c2_ref[...] = jnp.sum(cbt * cbt, axis=0, keepdims=True)


def _mm_t(a, b):
    return jax.lax.dot_general(
        a, b, (((1,), (1,)), ((), ())),
        preferred_element_type=jnp.float32)


def _vq_kernel(x_ref, xt_ref, ch_ref, clh_ref, cbh_ref, cbm_ref, cbl_ref,
               c2_ref, l2_ref, codes_ref, vec_ref):
    x = x_ref[...]
    xh, xl = _split2(x)
    xhl = jnp.concatenate([xh, xl], axis=1)
    cross = _mm(xh, ch_ref[...]) + _mm(xhl, clh_ref[...])
    e = c2_ref[...] + cross
    x2 = jnp.sum(x * x, axis=1, keepdims=True)
    l2_ref[...] = x2 + e

    iota = jax.lax.broadcasted_iota(jnp.int32, (1, N_CODES), 1)
    am1 = jnp.argmin(e, axis=1).astype(jnp.int32)
    oh1 = am1[:, None] == iota
    e2 = jnp.where(oh1, jnp.float32(jnp.inf), e)
    am2 = jnp.argmin(e2, axis=1).astype(jnp.int32)
    oh2 = am2[:, None] == iota

    def gather_exact_t(oh):
        ohb = oh.astype(jnp.bfloat16)
        return (_mm_t(cbh_ref[...], ohb) + _mm_t(cbm_ref[...], ohb)
                + _mm_t(cbl_ref[...], ohb))

    xt = xt_ref[...]
    g0 = gather_exact_t(oh1)
    g1 = gather_exact_t(oh2)
    d0 = _ref_order_sum_t((g0 - xt) * (g0 - xt))
    d1 = _ref_order_sum_t((g1 - xt) * (g1 - xt))
    a1 = am1[None, :]
    a2 = am2[None, :]
    take2 = (d1 < d0) | ((d1 == d0) & (a2 < a1))
    codes_ref[...] = jnp.where(take2, a2, a1)[0, :]
    vec_ref[...] = jnp.where(take2, g1, g0).T


@functools.partial(jax.jit, static_argnames=())
def kernel(x, codebook):
    bf16 = jnp.bfloat16
    f32 = jnp.float32
    cbt2 = -2.0 * codebook.T
    ch, cl, cbh, cbm, cbl, c2 = pl.pallas_call(
        _prep_kernel,
        out_shape=[
            jax.ShapeDtypeStruct((DIM, N_CODES), bf16),
            jax.ShapeDtypeStruct((2 * DIM, N_CODES), bf16),
            jax.ShapeDtypeStruct((DIM, N_CODES), bf16),
            jax.ShapeDtypeStruct((DIM, N_CODES), bf16),
            jax.ShapeDtypeStruct((DIM, N_CODES), bf16),
            jax.ShapeDtypeStruct((1, N_CODES), f32),
        ],
    )(cbt2, codebook)
    xt = x.T

    grid = (N_TOK // BLK,)
    l2, codes, vec = pl.pallas_call(
        _vq_kernel,
        grid=grid,
        in_specs=[
            pl.BlockSpec((BLK, DIM), lambda i: (i, 0)),
            pl.BlockSpec((DIM, BLK), lambda i: (0, i)),
            pl.BlockSpec((DIM, N_CODES), lambda i: (0, 0)),
            pl.BlockSpec((2 * DIM, N_CODES), lambda i: (0, 0)),
            pl.BlockSpec((DIM, N_CODES), lambda i: (0, 0)),
            pl.BlockSpec((DIM, N_CODES), lambda i: (0, 0)),
            pl.BlockSpec((DIM, N_CODES), lambda i: (0, 0)),
            pl.BlockSpec((1, N_CODES), lambda i: (0, 0)),
        ],
        out_specs=[
            pl.BlockSpec((BLK, N_CODES), lambda i: (i, 0)),
            pl.BlockSpec((BLK,), lambda i: (i,)),
            pl.BlockSpec((BLK, DIM), lambda i: (i, 0)),
        ],
        out_shape=[
            jax.ShapeDtypeStruct((N_TOK, N_CODES), f32),
            jax.ShapeDtypeStruct((N_TOK,), jnp.int32),
            jax.ShapeDtypeStruct((N_TOK, DIM), f32),
        ],
        compiler_params=pltpu.CompilerParams(
            dimension_semantics=("parallel",)),
    )(x, xt, ch, cl, cbh, cbm, cbl, c2)
    return (vec, codes, l2)


# speedup vs baseline: 1.5231x; 1.0611x over previous
import functools

import jax
import jax.numpy as jnp
from jax.experimental import pallas as pl
from jax.experimental.pallas import tpu as pltpu

N_TOK = 36864
N_CODES = 1024
DIM = 64
BLK = 2048


def _split2(a):
    hi = a.astype(jnp.bfloat16)
    lo = (a - hi.astype(jnp.float32)).astype(jnp.bfloat16)
    return hi, lo


def _split3(a):
    hi = a.astype(jnp.bfloat16)
    r = a - hi.astype(jnp.float32)
    mid = r.astype(jnp.bfloat16)
    lo = (r - mid.astype(jnp.float32)).astype(jnp.bfloat16)
    return hi, mid, lo


def _mm(a, b):
    return jax.lax.dot_general(
        a, b, (((1,), (0,)), ((), ())),
        preferred_element_type=jnp.float32)


def _ref_order_sum_t(sq):
    t = sq[0:8, :]
    for i in range(1, 8):
        t = t + sq[8 * i:8 * i + 8, :]
    u = t[0:4, :] + t[4:8, :]
    w = u[0:2, :] + u[2:4, :]
    return w[0:1, :] + w[1:2, :]


def _prep_kernel(cbt2_ref, cb_ref, ch_ref, clh_ref,
                 cbh_ref, cbm_ref, cbl_ref, c2_ref):
    ch, cl = _split2(cbt2_ref[...])
    ch_ref[...] = ch
    clh_ref[0:DIM, :] = cl
    clh_ref[DIM:2 * DIM, :] = ch
    cbt = cbt2_ref[...] * -0.5
    cbh, cbm, cbl = _split3(cbt)
    cbh_ref[...] = cbh
    cbm_ref[...] = cbm
    cbl_ref[...] = cbl
    c2_ref[...] = jnp.sum(cbt * cbt, axis=0, keepdims=True)


def _mm_t(a, b):
    return jax.lax.dot_general(
        a, b, (((1,), (1,)), ((), ())),
        preferred_element_type=jnp.float32)


def _vq_kernel(x_ref, xt_ref, ch_ref, clh_ref, cbh_ref, cbm_ref, cbl_ref,
               c2_ref, l2_ref, codes_ref, vec_ref):
    x = x_ref[...]
    xh, xl = _split2(x)
    xhl = jnp.concatenate([xh, xl], axis=1)
    cross = _mm(xh, ch_ref[...]) + _mm(xhl, clh_ref[...])
    e = c2_ref[...] + cross
    x2 = jnp.sum(x * x, axis=1, keepdims=True)
    l2_ref[...] = x2 + e

    iota = jax.lax.broadcasted_iota(jnp.int32, (1, N_CODES), 1)
    am1 = jnp.argmin(e, axis=1).astype(jnp.int32)
    oh1 = am1[:, None] == iota
    e2 = jnp.where(oh1, jnp.float32(jnp.inf), e)
    am2 = jnp.argmin(e2, axis=1).astype(jnp.int32)
    oh2 = am2[:, None] == iota

    def gather_exact_t(oh):
        ohb = oh.astype(jnp.bfloat16)
        return (_mm_t(cbh_ref[...], ohb) + _mm_t(cbm_ref[...], ohb)
                + _mm_t(cbl_ref[...], ohb))

    xt = xt_ref[...]
    g0 = gather_exact_t(oh1)
    g1 = gather_exact_t(oh2)
    d0 = _ref_order_sum_t((g0 - xt) * (g0 - xt))
    d1 = _ref_order_sum_t((g1 - xt) * (g1 - xt))
    a1 = am1[None, :]
    a2 = am2[None, :]
    take2 = (d1 < d0) | ((d1 == d0) & (a2 < a1))
    codes_ref[...] = jnp.where(take2, a2, a1)[0, :]
    vec_ref[...] = jnp.where(take2, g1, g0).T


@functools.partial(jax.jit, static_argnames=())
def kernel(x, codebook):
    bf16 = jnp.bfloat16
    f32 = jnp.float32
    cbt2 = -2.0 * codebook.T
    ch, cl, cbh, cbm, cbl, c2 = pl.pallas_call(
        _prep_kernel,
        out_shape=[
            jax.ShapeDtypeStruct((DIM, N_CODES), bf16),
            jax.ShapeDtypeStruct((2 * DIM, N_CODES), bf16),
            jax.ShapeDtypeStruct((DIM, N_CODES), bf16),
            jax.ShapeDtypeStruct((DIM, N_CODES), bf16),
            jax.ShapeDtypeStruct((DIM, N_CODES), bf16),
            jax.ShapeDtypeStruct((1, N_CODES), f32),
        ],
    )(cbt2, codebook)
    xt = x.T

    grid = (N_TOK // BLK,)
    l2, codes, vec = pl.pallas_call(
        _vq_kernel,
        grid=grid,
        in_specs=[
            pl.BlockSpec((BLK, DIM), lambda i: (i, 0)),
            pl.BlockSpec((DIM, BLK), lambda i: (0, i)),
            pl.BlockSpec((DIM, N_CODES), lambda i: (0, 0)),
            pl.BlockSpec((2 * DIM, N_CODES), lambda i: (0, 0)),
            pl.BlockSpec((DIM, N_CODES), lambda i: (0, 0)),
            pl.BlockSpec((DIM, N_CODES), lambda i: (0, 0)),
            pl.BlockSpec((DIM, N_CODES), lambda i: (0, 0)),
            pl.BlockSpec((1, N_CODES), lambda i: (0, 0)),
        ],
        out_specs=[
            pl.BlockSpec((BLK, N_CODES), lambda i: (i, 0)),
            pl.BlockSpec((BLK,), lambda i: (i,)),
            pl.BlockSpec((BLK, DIM), lambda i: (i, 0)),
        ],
        out_shape=[
            jax.ShapeDtypeStruct((N_TOK, N_CODES), f32),
            jax.ShapeDtypeStruct((N_TOK,), jnp.int32),
            jax.ShapeDtypeStruct((N_TOK, DIM), f32),
        ],
        compiler_params=pltpu.CompilerParams(
            dimension_semantics=("parallel",)),
    )(x, xt, ch, cl, cbh, cbm, cbl, c2)
    return (vec, codes, l2)


# speedup vs baseline: 1.5521x; 1.0190x over previous
import functools

import jax
import jax.numpy as jnp
from jax.experimental import pallas as pl
from jax.experimental.pallas import tpu as pltpu

N_TOK = 36864
N_CODES = 1024
DIM = 64
BLK = 3072


def _split2(a):
    hi = a.astype(jnp.bfloat16)
    lo = (a - hi.astype(jnp.float32)).astype(jnp.bfloat16)
    return hi, lo


def _split3(a):
    hi = a.astype(jnp.bfloat16)
    r = a - hi.astype(jnp.float32)
    mid = r.astype(jnp.bfloat16)
    lo = (r - mid.astype(jnp.float32)).astype(jnp.bfloat16)
    return hi, mid, lo


def _mm(a, b):
    return jax.lax.dot_general(
        a, b, (((1,), (0,)), ((), ())),
        preferred_element_type=jnp.float32)


def _ref_order_sum_t(sq):
    t = sq[0:8, :]
    for i in range(1, 8):
        t = t + sq[8 * i:8 * i + 8, :]
    u = t[0:4, :] + t[4:8, :]
    w = u[0:2, :] + u[2:4, :]
    return w[0:1, :] + w[1:2, :]


def _prep_kernel(cbt2_ref, cb_ref, ch_ref, clh_ref,
                 cbh_ref, cbm_ref, cbl_ref, c2_ref):
    ch, cl = _split2(cbt2_ref[...])
    ch_ref[...] = ch
    clh_ref[0:DIM, :] = cl
    clh_ref[DIM:2 * DIM, :] = ch
    cbt = cbt2_ref[...] * -0.5
    cbh, cbm, cbl = _split3(cbt)
    cbh_ref[...] = cbh
    cbm_ref[...] = cbm
    cbl_ref[...] = cbl
    c2_ref[...] = jnp.sum(cbt * cbt, axis=0, keepdims=True)


def _mm_t(a, b):
    return jax.lax.dot_general(
        a, b, (((1,), (1,)), ((), ())),
        preferred_element_type=jnp.float32)


def _vq_kernel(x_ref, xt_ref, ch_ref, clh_ref, cbh_ref, cbm_ref, cbl_ref,
               c2_ref, l2_ref, codes_ref, vec_ref):
    x = x_ref[...]
    xh, xl = _split2(x)
    xhl = jnp.concatenate([xh, xl], axis=1)
    cross = _mm(xh, ch_ref[...]) + _mm(xhl, clh_ref[...])
    e = c2_ref[...] + cross
    x2 = jnp.sum(x * x, axis=1, keepdims=True)
    l2_ref[...] = x2 + e

    iota = jax.lax.broadcasted_iota(jnp.int32, (1, N_CODES), 1)
    am1 = jnp.argmin(e, axis=1).astype(jnp.int32)
    oh1 = am1[:, None] == iota
    e2 = jnp.where(oh1, jnp.float32(jnp.inf), e)
    am2 = jnp.argmin(e2, axis=1).astype(jnp.int32)
    oh2 = am2[:, None] == iota

    def gather_exact_t(oh):
        ohb = oh.astype(jnp.bfloat16)
        return (_mm_t(cbh_ref[...], ohb) + _mm_t(cbm_ref[...], ohb)
                + _mm_t(cbl_ref[...], ohb))

    xt = xt_ref[...]
    g0 = gather_exact_t(oh1)
    g1 = gather_exact_t(oh2)
    d0 = _ref_order_sum_t((g0 - xt) * (g0 - xt))
    d1 = _ref_order_sum_t((g1 - xt) * (g1 - xt))
    a1 = am1[None, :]
    a2 = am2[None, :]
    take2 = (d1 < d0) | ((d1 == d0) & (a2 < a1))
    codes_ref[...] = jnp.where(take2, a2, a1)[0, :]
    vec_ref[...] = jnp.where(take2, g1, g0).T


@functools.partial(jax.jit, static_argnames=())
def kernel(x, codebook):
    bf16 = jnp.bfloat16
    f32 = jnp.float32
    cbt2 = -2.0 * codebook.T
    ch, cl, cbh, cbm, cbl, c2 = pl.pallas_call(
        _prep_kernel,
        out_shape=[
            jax.ShapeDtypeStruct((DIM, N_CODES), bf16),
            jax.ShapeDtypeStruct((2 * DIM, N_CODES), bf16),
            jax.ShapeDtypeStruct((DIM, N_CODES), bf16),
            jax.ShapeDtypeStruct((DIM, N_CODES), bf16),
            jax.ShapeDtypeStruct((DIM, N_CODES), bf16),
            jax.ShapeDtypeStruct((1, N_CODES), f32),
        ],
    )(cbt2, codebook)
    xt = x.T

    grid = (N_TOK // BLK,)
    l2, codes, vec = pl.pallas_call(
        _vq_kernel,
        grid=grid,
        in_specs=[
            pl.BlockSpec((BLK, DIM), lambda i: (i, 0)),
            pl.BlockSpec((DIM, BLK), lambda i: (0, i)),
            pl.BlockSpec((DIM, N_CODES), lambda i: (0, 0)),
            pl.BlockSpec((2 * DIM, N_CODES), lambda i: (0, 0)),
            pl.BlockSpec((DIM, N_CODES), lambda i: (0, 0)),
            pl.BlockSpec((DIM, N_CODES), lambda i: (0, 0)),
            pl.BlockSpec((DIM, N_CODES), lambda i: (0, 0)),
            pl.BlockSpec((1, N_CODES), lambda i: (0, 0)),
        ],
        out_specs=[
            pl.BlockSpec((BLK, N_CODES), lambda i: (i, 0)),
            pl.BlockSpec((BLK,), lambda i: (i,)),
            pl.BlockSpec((BLK, DIM), lambda i: (i, 0)),
        ],
        out_shape=[
            jax.ShapeDtypeStruct((N_TOK, N_CODES), f32),
            jax.ShapeDtypeStruct((N_TOK,), jnp.int32),
            jax.ShapeDtypeStruct((N_TOK, DIM), f32),
        ],
        compiler_params=pltpu.CompilerParams(
            dimension_semantics=("parallel",)),
    )(x, xt, ch, cl, cbh, cbm, cbl, c2)
    return (vec, codes, l2)
